# Initial kernel scaffold; baseline (speedup 1.0000x reference)
import jax
import jax.numpy as jnp
from jax.experimental import pallas as pl


def kernel(x, edge_index):
    raise NotImplementedError("write your pallas kernel here")


# speedup vs baseline: 5.2444x; 5.2444x over previous
import functools

import jax
import jax.numpy as jnp
from jax import lax
from jax.experimental import pallas as pl
from jax.experimental.pallas import tpu as pltpu
from jax.experimental.pallas import tpu_sc as plsc

N = 10000
NPAD = 10240
D = 128
DH = D // 2
E = 320000
EPT = E // 16
B = 80
STEPS = EPT // B
ROWS = NPAD // 16


def _gcn_kernel(x_hbm, src_hbm, dst_hbm, out_hbm,
                sx, sacc, sdeg,
                src_idx, dst_idx, rows, ones_v, zeros_v, acc_v, inv_v,
                sem):
    c = lax.axis_index("c")
    s = lax.axis_index("s")
    nb = s * ROWS

    zero16 = jnp.zeros((16,), jnp.float32)
    one16 = jnp.ones((16,), jnp.float32)

    def _zero_acc(i, _):
        for k in range(4):
            acc_v[i, pl.ds(16 * k, 16)] = zero16
        return 0
    lax.fori_loop(0, ROWS, _zero_acc, 0)

    def _zero_z(i, _):
        zeros_v[pl.ds(16 * i, 16)] = zero16
        return 0
    lax.fori_loop(0, ROWS // 16, _zero_z, 0)

    for i in range(B // 16):
        ones_v[pl.ds(16 * i, 16)] = one16

    pltpu.sync_copy(acc_v, sacc.at[pl.ds(nb, ROWS), :])
    pltpu.sync_copy(zeros_v, sdeg.at[pl.ds(nb, ROWS)])

    @pl.when(s == 0)
    def _stage_x():
        pltpu.sync_copy(x_hbm.at[:, pl.ds(c * DH, DH)], sx.at[pl.ds(0, N), :])

    plsc.subcore_barrier()

    def _step(i, _):
        eb = s * EPT + i * B
        pltpu.sync_copy(src_hbm.at[pl.ds(eb, B)], src_idx)
        pltpu.sync_copy(dst_hbm.at[pl.ds(eb, B)], dst_idx)
        pltpu.async_copy(sx.at[src_idx], rows, sem).wait()
        pltpu.sync_copy(rows, sacc.at[dst_idx], add=True)
        pltpu.sync_copy(ones_v, sdeg.at[dst_idx], add=True)
        return 0
    lax.fori_loop(0, STEPS, _step, 0)

    plsc.subcore_barrier()

    pltpu.sync_copy(sacc.at[pl.ds(nb, ROWS), :], acc_v)
    pltpu.sync_copy(sdeg.at[pl.ds(nb, ROWS)], zeros_v)

    def _inv(j, _):
        d = zeros_v[pl.ds(16 * j, 16)]
        inv_v[pl.ds(16 * j, 16)] = 1.0 / jnp.maximum(d, 1.0)
        return 0
    lax.fori_loop(0, ROWS // 16, _inv, 0)

    def _scale(i, _):
        sc = inv_v[pl.ds(i, 16)][0]
        for k in range(4):
            acc_v[i, pl.ds(16 * k, 16)] = acc_v[i, pl.ds(16 * k, 16)] * sc
        return 0
    lax.fori_loop(0, ROWS, _scale, 0)

    pltpu.sync_copy(acc_v, out_hbm.at[pl.ds(nb, ROWS), pl.ds(c * DH, DH)])


@jax.jit
def _gcn(x, src, dst):
    mesh = plsc.VectorSubcoreMesh(core_axis_name="c", subcore_axis_name="s")
    run = pl.kernel(
        _gcn_kernel,
        out_type=jax.ShapeDtypeStruct((NPAD, D), jnp.float32),
        mesh=mesh,
        scratch_types=[
            pltpu.VMEM_SHARED((NPAD, DH), jnp.float32),
            pltpu.VMEM_SHARED((NPAD, DH), jnp.float32),
            pltpu.VMEM_SHARED((NPAD,), jnp.float32),
            pltpu.VMEM((B,), jnp.int32),
            pltpu.VMEM((B,), jnp.int32),
            pltpu.VMEM((B, DH), jnp.float32),
            pltpu.VMEM((B,), jnp.float32),
            pltpu.VMEM((ROWS,), jnp.float32),
            pltpu.VMEM((ROWS, DH), jnp.float32),
            pltpu.VMEM((ROWS + 16,), jnp.float32),
            pltpu.SemaphoreType.DMA,
        ],
        compiler_params=pltpu.CompilerParams(use_tc_tiling_on_sc=False),
    )
    return run(x, src, dst)


def kernel(x, edge_index):
    src = edge_index[0]
    dst = edge_index[1]
    out = _gcn(x, src, dst)
    return out[:N]


# speedup vs baseline: 10.0516x; 1.9166x over previous
import functools

import jax
import jax.numpy as jnp
from jax import lax
from jax.experimental import pallas as pl
from jax.experimental.pallas import tpu as pltpu
from jax.experimental.pallas import tpu_sc as plsc

N = 10000
NPAD = 10240
D = 128
DH = D // 2
E = 320000
EPT = E // 16
B = 400
STEPS = EPT // B
ROWS = NPAD // 16
RCHUNK = 160


def _gcn_kernel(x_hbm, src_hbm, dst_hbm, out_hbm,
                sx, sacc, sdeg,
                src_idx, dst_idx, rows, ones_v, zeros_v, acc_v, inv_v,
                sem):
    c = lax.axis_index("c")
    s = lax.axis_index("s")
    nb = s * ROWS

    zero16 = jnp.zeros((16,), jnp.float32)
    one16 = jnp.ones((16,), jnp.float32)

    def _zero_acc(i, _):
        for k in range(4):
            acc_v[i, pl.ds(16 * k, 16)] = zero16
        return 0
    lax.fori_loop(0, RCHUNK, _zero_acc, 0)

    def _zero_z(i, _):
        zeros_v[pl.ds(16 * i, 16)] = zero16
        return 0
    lax.fori_loop(0, ROWS // 16, _zero_z, 0)

    for i in range(B // 16):
        ones_v[pl.ds(16 * i, 16)] = one16

    for j in range(ROWS // RCHUNK):
        pltpu.sync_copy(acc_v, sacc.at[pl.ds(nb + j * RCHUNK, RCHUNK), :])
    pltpu.sync_copy(zeros_v, sdeg.at[pl.ds(nb, ROWS)])

    @pl.when(s == 0)
    def _stage_x():
        pltpu.sync_copy(x_hbm.at[:, pl.ds(c * DH, DH)], sx.at[pl.ds(0, N), :])

    plsc.subcore_barrier()

    def _step(i, _):
        eb = s * EPT + i * B
        pltpu.sync_copy(src_hbm.at[pl.ds(eb, B)], src_idx)
        pltpu.sync_copy(dst_hbm.at[pl.ds(eb, B)], dst_idx)
        pltpu.async_copy(sx.at[src_idx], rows, sem).wait()
        pltpu.sync_copy(rows, sacc.at[dst_idx], add=True)
        pltpu.sync_copy(ones_v, sdeg.at[dst_idx], add=True)
        return 0
    lax.fori_loop(0, STEPS, _step, 0)

    plsc.subcore_barrier()

    pltpu.sync_copy(sdeg.at[pl.ds(nb, ROWS)], zeros_v)

    def _inv(j, _):
        d = zeros_v[pl.ds(16 * j, 16)]
        inv_v[pl.ds(16 * j, 16)] = 1.0 / jnp.maximum(d, 1.0)
        return 0
    lax.fori_loop(0, ROWS // 16, _inv, 0)

    for j in range(ROWS // RCHUNK):
        rb = nb + j * RCHUNK
        pltpu.sync_copy(sacc.at[pl.ds(rb, RCHUNK), :], acc_v)

        def _scale(i, _):
            sc = inv_v[pl.ds(j * RCHUNK + i, 16)][0]
            for k in range(4):
                acc_v[i, pl.ds(16 * k, 16)] = acc_v[i, pl.ds(16 * k, 16)] * sc
            return 0
        lax.fori_loop(0, RCHUNK, _scale, 0)

        pltpu.sync_copy(acc_v, out_hbm.at[pl.ds(rb, RCHUNK), pl.ds(c * DH, DH)])


@jax.jit
def _gcn(x, src, dst):
    mesh = plsc.VectorSubcoreMesh(core_axis_name="c", subcore_axis_name="s")
    run = pl.kernel(
        _gcn_kernel,
        out_type=jax.ShapeDtypeStruct((NPAD, D), jnp.float32),
        mesh=mesh,
        scratch_types=[
            pltpu.VMEM_SHARED((NPAD, DH), jnp.float32),
            pltpu.VMEM_SHARED((NPAD, DH), jnp.float32),
            pltpu.VMEM_SHARED((NPAD,), jnp.float32),
            pltpu.VMEM((B,), jnp.int32),
            pltpu.VMEM((B,), jnp.int32),
            pltpu.VMEM((B, DH), jnp.float32),
            pltpu.VMEM((B,), jnp.float32),
            pltpu.VMEM((ROWS,), jnp.float32),
            pltpu.VMEM((RCHUNK, DH), jnp.float32),
            pltpu.VMEM((ROWS + 16,), jnp.float32),
            pltpu.SemaphoreType.DMA,
        ],
        compiler_params=pltpu.CompilerParams(use_tc_tiling_on_sc=False),
    )
    return run(x, src, dst)


def kernel(x, edge_index):
    src = edge_index[0]
    dst = edge_index[1]
    out = _gcn(x, src, dst)
    return out[:N]


# speedup vs baseline: 15.8817x; 1.5800x over previous
import jax
import jax.numpy as jnp
from jax import lax
from jax.experimental import pallas as pl
from jax.experimental.pallas import tpu as pltpu
from jax.experimental.pallas import tpu_sc as plsc

N = 10000
NPAD = 10240
D = 128
DH = D // 2
E = 320000
EPT = E // 16
B = 200
NBUF = 4
DCH = 4000
SPC = DCH // B
NCH = EPT // DCH
TSTEPS = EPT // B
ROWS = NPAD // 16
RCHUNK = 80
NRCH = ROWS // RCHUNK


def _gcn_kernel(srcL_hbm, srcR_hbm, x2_hbm, dst_hbm, out_hbm,
                sacc, sdeg,
                src_c0, src_c1, dst_c0, dst_c1,
                rows0, rows1, rows2, rows3,
                ones4k, zeros_v, acc_v0, acc_v1, inv_v,
                sem_g0, sem_g1, sem_g2, sem_g3,
                sem_s0, sem_s1, sem_s2, sem_s3,
                sem_i, sem_d, sem_a0, sem_a1, sem_o0, sem_o1):
    c = lax.axis_index("c")
    s = lax.axis_index("s")
    nb = s * ROWS

    src_c = (src_c0, src_c1)
    dst_c = (dst_c0, dst_c1)
    rows = (rows0, rows1, rows2, rows3)
    sem_g = (sem_g0, sem_g1, sem_g2, sem_g3)
    sem_s = (sem_s0, sem_s1, sem_s2, sem_s3)

    zero16 = jnp.zeros((16,), jnp.float32)
    one16 = jnp.ones((16,), jnp.float32)

    def _zero_acc(i, _):
        for k in range(4):
            acc_v0[i, pl.ds(16 * k, 16)] = zero16
        return 0
    lax.fori_loop(0, RCHUNK, _zero_acc, 0)

    def _zero_z(i, _):
        zeros_v[pl.ds(16 * i, 16)] = zero16
        return 0
    lax.fori_loop(0, ROWS // 16, _zero_z, 0)

    def _ones4k(i, _):
        ones4k[pl.ds(16 * i, 16)] = one16
        return 0
    lax.fori_loop(0, DCH // 16, _ones4k, 0)

    zdescs = [pltpu.async_copy(acc_v0,
                               sacc.at[pl.ds(nb + j * RCHUNK, RCHUNK), :],
                               sem_a0)
              for j in range(NRCH)]
    zdescs.append(pltpu.async_copy(zeros_v, sdeg.at[pl.ds(nb, ROWS)], sem_a1))
    for dsc in zdescs:
        dsc.wait()

    plsc.subcore_barrier()

    def issue_idx(k):
        p = k % 2
        eb = s * EPT + k * DCH

        @pl.when(c == 0)
        def _cl():
            pltpu.async_copy(srcL_hbm.at[pl.ds(eb, DCH)], src_c[p], sem_i)

        @pl.when(c == 1)
        def _cr():
            pltpu.async_copy(srcR_hbm.at[pl.ds(eb, DCH)], src_c[p], sem_i)

        pltpu.async_copy(dst_hbm.at[pl.ds(eb, DCH)], dst_c[p], sem_i)

    def wait_idx(k):
        p = k % 2
        eb = s * EPT + k * DCH
        pltpu.make_async_copy(srcL_hbm.at[pl.ds(eb, DCH)], src_c[p],
                              sem_i).wait()
        pltpu.make_async_copy(dst_hbm.at[pl.ds(eb, DCH)], dst_c[p],
                              sem_i).wait()

    desc_g = [None] * TSTEPS
    desc_s = [None] * TSTEPS
    desc_d = [None] * NCH

    def start_gather(t):
        k, j = divmod(t, SPC)
        desc_g[t] = pltpu.async_copy(
            x2_hbm.at[src_c[k % 2].at[pl.ds(j * B, B)]],
            rows[t % NBUF], sem_g[t % NBUF])

    def scatter(t):
        k, j = divmod(t, SPC)
        desc_g[t].wait()
        desc_s[t] = pltpu.async_copy(
            rows[t % NBUF], sacc.at[dst_c[k % 2].at[pl.ds(j * B, B)]],
            sem_s[t % NBUF], add=True)

    issue_idx(0)
    wait_idx(0)
    desc_d[0] = pltpu.async_copy(ones4k, sdeg.at[dst_c[0]], sem_d, add=True)

    for t in range(TSTEPS):
        k, j = divmod(t, SPC)
        if j == 0 and k > 0:
            wait_idx(k)
            desc_d[k] = pltpu.async_copy(ones4k, sdeg.at[dst_c[k % 2]],
                                         sem_d, add=True)
        if j == 4 and k < NCH - 1:
            if k > 0:
                desc_d[k - 1].wait()
            issue_idx(k + 1)
        if t >= NBUF:
            desc_s[t - NBUF].wait()
        start_gather(t)
        if t >= 2:
            scatter(t - 2)

    scatter(TSTEPS - 2)
    scatter(TSTEPS - 1)
    for t in range(TSTEPS - NBUF, TSTEPS):
        desc_s[t].wait()
    desc_d[NCH - 2].wait()
    desc_d[NCH - 1].wait()

    plsc.subcore_barrier()

    pltpu.sync_copy(sdeg.at[pl.ds(nb, ROWS)], zeros_v)

    def _inv(j, _):
        d = zeros_v[pl.ds(16 * j, 16)]
        inv_v[pl.ds(16 * j, 16)] = 1.0 / jnp.maximum(d, 1.0)
        return 0
    lax.fori_loop(0, ROWS // 16, _inv, 0)

    acc_v = (acc_v0, acc_v1)
    sem_a = (sem_a0, sem_a1)
    sem_o = (sem_o0, sem_o1)
    d_in = [None] * NRCH
    d_out = [None] * NRCH

    def _in_copy(j):
        return pltpu.async_copy(sacc.at[pl.ds(nb + j * RCHUNK, RCHUNK), :],
                                acc_v[j % 2], sem_a[j % 2])

    d_in[0] = _in_copy(0)
    for j in range(NRCH):
        b = j % 2
        if j + 1 < NRCH:
            if j >= 1:
                d_out[j - 1].wait()
            d_in[j + 1] = _in_copy(j + 1)
        d_in[j].wait()

        def _scale(g, _):
            iv = inv_v[pl.ds(j * RCHUNK + 16 * g, 16)]
            for l in range(16):
                i = 16 * g + l
                sc = iv[l]
                for k in range(4):
                    acc_v[b][i, pl.ds(16 * k, 16)] = (
                        acc_v[b][i, pl.ds(16 * k, 16)] * sc)
            return 0
        lax.fori_loop(0, RCHUNK // 16, _scale, 0)

        d_out[j] = pltpu.async_copy(
            acc_v[b],
            out_hbm.at[pl.ds(nb + j * RCHUNK, RCHUNK), pl.ds(c * DH, DH)],
            sem_o[b])
    d_out[NRCH - 2].wait()
    d_out[NRCH - 1].wait()


@jax.jit
def _gcn(x, src, dst):
    mesh = plsc.VectorSubcoreMesh(core_axis_name="c", subcore_axis_name="s")
    x2 = x.reshape(2 * N, DH)
    srcL = src * 2
    srcR = srcL + 1
    run = pl.kernel(
        _gcn_kernel,
        out_type=jax.ShapeDtypeStruct((NPAD, D), jnp.float32),
        mesh=mesh,
        scratch_types=[
            pltpu.VMEM_SHARED((NPAD, DH), jnp.float32),
            pltpu.VMEM_SHARED((NPAD,), jnp.float32),
            pltpu.VMEM((DCH,), jnp.int32),
            pltpu.VMEM((DCH,), jnp.int32),
            pltpu.VMEM((DCH,), jnp.int32),
            pltpu.VMEM((DCH,), jnp.int32),
            pltpu.VMEM((B, DH), jnp.float32),
            pltpu.VMEM((B, DH), jnp.float32),
            pltpu.VMEM((B, DH), jnp.float32),
            pltpu.VMEM((B, DH), jnp.float32),
            pltpu.VMEM((DCH,), jnp.float32),
            pltpu.VMEM((ROWS,), jnp.float32),
            pltpu.VMEM((RCHUNK, DH), jnp.float32),
            pltpu.VMEM((RCHUNK, DH), jnp.float32),
            pltpu.VMEM((ROWS + 16,), jnp.float32),
            pltpu.SemaphoreType.DMA,
            pltpu.SemaphoreType.DMA,
            pltpu.SemaphoreType.DMA,
            pltpu.SemaphoreType.DMA,
            pltpu.SemaphoreType.DMA,
            pltpu.SemaphoreType.DMA,
            pltpu.SemaphoreType.DMA,
            pltpu.SemaphoreType.DMA,
            pltpu.SemaphoreType.DMA,
            pltpu.SemaphoreType.DMA,
            pltpu.SemaphoreType.DMA,
            pltpu.SemaphoreType.DMA,
            pltpu.SemaphoreType.DMA,
            pltpu.SemaphoreType.DMA,
        ],
        compiler_params=pltpu.CompilerParams(use_tc_tiling_on_sc=False),
    )
    return run(srcL, srcR, x2, dst)


def kernel(x, edge_index):
    src = edge_index[0]
    dst = edge_index[1]
    out = _gcn(x, src, dst)
    return out[:N]
